# baseline (speedup 1.0000x reference)
import jax
import jax.numpy as jnp
from jax.experimental import pallas as pl
from jax.experimental.pallas import tpu as pltpu

_B, _N, _F, _CH = 32, 200, 128, 32
_H1, _H2 = 512, 256
_NK = 4
_KC = (_N * _CH) // _NK


def _elu(v):
    return jnp.where(v > 0, v, jnp.exp(jnp.minimum(v, 0.0)) - 1.0)


def _net_kernel(x_ref, a_ref, w1_ref, b1t_ref, w2_ref, b2_ref,
                wf1_ref, bf1_ref, wf2_ref, bf2_ref, wf3_ref, bf3_ref,
                o_ref, flat_s, acc_s):
    k = pl.program_id(0)

    @pl.when(k == 0)
    def _gcn():
        bf = jnp.bfloat16
        xw = jnp.dot(x_ref[...].astype(bf), w1_ref[...].astype(bf),
                     preferred_element_type=jnp.float32)
        y = jnp.concatenate(
            [xw[b * _N:(b + 1) * _N, :] for b in range(_B)], axis=1)
        a = a_ref[...].astype(bf)
        h1 = _elu(jnp.dot(a, y.astype(bf), preferred_element_type=jnp.float32)
                  + b1t_ref[...])
        u = jnp.dot(a, h1.astype(bf), preferred_element_type=jnp.float32)
        v = jnp.concatenate(
            [u[:, b * _CH:(b + 1) * _CH] for b in range(_B)], axis=0)
        h2 = _elu(jnp.dot(v.astype(bf), w2_ref[...].astype(bf),
                          preferred_element_type=jnp.float32)
                  + b2_ref[...])
        s3 = h2.reshape(_B, _N, _CH)
        t1 = jnp.transpose(s3, (1, 0, 2))
        t2 = jnp.transpose(t1, (0, 2, 1))
        flat_s[...] = t2.reshape(_N * _CH, _B).T
        acc_s[...] = jnp.zeros_like(acc_s)

    acc_s[...] += jnp.dot(flat_s[:, pl.ds(pl.multiple_of(k * _KC, 128), _KC)], wf1_ref[...],
                          preferred_element_type=jnp.float32)

    @pl.when(k == _NK - 1)
    def _tail():
        t = jax.nn.relu(acc_s[...] + bf1_ref[...])
        t = jax.nn.relu(jnp.dot(t, wf2_ref[...],
                                preferred_element_type=jnp.float32)
                        + bf2_ref[...])
        o_ref[...] = jax.nn.sigmoid(
            jnp.dot(t, wf3_ref[...], preferred_element_type=jnp.float32)
            + bf3_ref[...])


def kernel(x, a, W1, b1, W2, b2, Wf1, bf1, Wf2, bf2, Wf3, bf3):
    x2 = x.reshape(_B * _N, _F)
    b1t = jnp.tile(b1, _B).reshape(1, _B * _CH)

    out = pl.pallas_call(
        _net_kernel,
        grid=(_NK,),
        in_specs=[
            pl.BlockSpec((_B * _N, _F), lambda k: (0, 0)),
            pl.BlockSpec((_N, _N), lambda k: (0, 0)),
            pl.BlockSpec((_F, _CH), lambda k: (0, 0)),
            pl.BlockSpec((1, _B * _CH), lambda k: (0, 0)),
            pl.BlockSpec((_CH, _CH), lambda k: (0, 0)),
            pl.BlockSpec((1, _CH), lambda k: (0, 0)),
            pl.BlockSpec((_KC, _H1), lambda k: (k, 0)),
            pl.BlockSpec((1, _H1), lambda k: (0, 0)),
            pl.BlockSpec((_H1, _H2), lambda k: (0, 0)),
            pl.BlockSpec((1, _H2), lambda k: (0, 0)),
            pl.BlockSpec((_H2, 1), lambda k: (0, 0)),
            pl.BlockSpec((1, 1), lambda k: (0, 0)),
        ],
        out_specs=pl.BlockSpec((_B, 1), lambda k: (0, 0)),
        out_shape=jax.ShapeDtypeStruct((_B, 1), jnp.float32),
        scratch_shapes=[
            pltpu.VMEM((_B, _N * _CH), jnp.float32),
            pltpu.VMEM((_B, _H1), jnp.float32),
        ],
        compiler_params=pltpu.CompilerParams(
            dimension_semantics=("arbitrary",)),
    )(x2, a, W1, b1t, W2, b2.reshape(1, _CH), Wf1,
      bf1.reshape(1, _H1), Wf2, bf2.reshape(1, _H2), Wf3, bf3.reshape(1, 1))
    return out
